# Initial kernel scaffold; baseline (speedup 1.0000x reference)
import jax
import jax.numpy as jnp
from jax.experimental import pallas as pl


def kernel(inputs, emb_weight):
    raise NotImplementedError("write your pallas kernel here")


# speedup vs baseline: 1.6134x; 1.6134x over previous
import functools

import jax
import jax.numpy as jnp
from jax import lax
from jax.experimental import pallas as pl
from jax.experimental.pallas import tpu as pltpu
from jax.experimental.pallas import tpu_sc as plsc

V = 8192
D = 32
B = 16384
RB = 512
GB = 4096
NC = 2
NS = 16
L = 16
NW = NC * NS
BPW = B // NW
CH = 128
NCH = BPW // CH
VSL = V // NS


def _argmin_body(x_ref, et_ref, idx_ref, e2_ref, col_ref):
    i = pl.program_id(0)

    @pl.when(i == 0)
    def _():
        et0 = et_ref[...]
        e2_ref[...] = jnp.sum(et0 * et0, axis=0, keepdims=True)
        col_ref[...] = lax.broadcasted_iota(
            jnp.int32, (1, V), 1).astype(jnp.float32)

    xb = x_ref[...]
    et = et_ref[...]
    x2 = jnp.sum(xb * xb, axis=1, keepdims=True)
    t = x2 + e2_ref[...]
    mm = jnp.dot((xb * -2.0).astype(jnp.bfloat16), et.astype(jnp.bfloat16),
                 preferred_element_type=jnp.float32)
    d = t + mm
    acc_v = None
    acc_i = None
    for g in range(V // GB):
        dg = d[:, g * GB:(g + 1) * GB]
        mg = jnp.min(dg, axis=1, keepdims=True)
        colg = col_ref[:, g * GB:(g + 1) * GB]
        ig = jnp.min(jnp.where(dg == mg, colg, jnp.float32(1e9)),
                     axis=1, keepdims=True)
        if g == 0:
            acc_v, acc_i = mg, ig
        else:
            ar = acc_v.astype(jnp.bfloat16).astype(jnp.float32)
            take = mg < ar
            acc_i = jnp.where(take, ig, acc_i)
            acc_v = jnp.where(take, mg, ar)
    idx_ref[...] = acc_i.reshape(RB).astype(jnp.int32)


def _argmin_call(x, emb_t):
    return pl.pallas_call(
        _argmin_body,
        grid=(B // RB,),
        in_specs=[
            pl.BlockSpec((RB, D), lambda i: (i, 0)),
            pl.BlockSpec((D, V), lambda i: (0, 0)),
        ],
        out_specs=pl.BlockSpec((RB,), lambda i: (i,)),
        out_shape=jax.ShapeDtypeStruct((B,), jnp.int32),
        scratch_shapes=[
            pltpu.VMEM((1, V), jnp.float32),
            pltpu.VMEM((1, V), jnp.float32),
        ],
    )(x, emb_t)


def _sc_body(emb_hbm, idx3_hbm, zeros_hbm, ones_hbm,
             quant_hbm, counts_hbm,
             idx_v, rows_v, ones_v, shared_cnt, sem):
    c = lax.axis_index("c")
    s = lax.axis_index("s")
    wid = s * NC + c
    base = wid * BPW

    pltpu.sync_copy(idx3_hbm.at[wid], idx_v)
    pltpu.sync_copy(ones_hbm, ones_v)
    pltpu.sync_copy(zeros_hbm.at[pl.ds(s * VSL, VSL)],
                    shared_cnt.at[pl.ds(s * VSL, VSL)])

    handles = [
        pltpu.async_copy(emb_hbm.at[idx_v.at[j]],
                         rows_v.at[pl.ds(j * CH, CH)], sem)
        for j in range(NCH)
    ]
    for h in handles:
        h.wait()
    pltpu.sync_copy(rows_v, quant_hbm.at[pl.ds(base, BPW)])

    plsc.subcore_barrier()
    for j in range(NCH):
        pltpu.sync_copy(ones_v, shared_cnt.at[idx_v.at[j]], add=True)
    plsc.subcore_barrier()
    pltpu.sync_copy(shared_cnt.at[pl.ds(s * VSL, VSL)],
                    counts_hbm.at[pl.ds(c * V + s * VSL, VSL)])


def _sc_call(emb, idx):
    idx3 = idx.reshape(NW, NCH, CH)
    zeros = jnp.zeros((V, L), jnp.float32)
    ones = jnp.ones((CH, L), jnp.float32)
    mesh = plsc.VectorSubcoreMesh(core_axis_name="c", subcore_axis_name="s")
    f = pl.kernel(
        _sc_body,
        out_type=[
            jax.ShapeDtypeStruct((B, D), jnp.float32),
            jax.ShapeDtypeStruct((NC * V, L), jnp.float32),
        ],
        mesh=mesh,
        scratch_types=[
            pltpu.VMEM((NCH, CH), jnp.int32),
            pltpu.VMEM((BPW, D), jnp.float32),
            pltpu.VMEM((CH, L), jnp.float32),
            pltpu.VMEM_SHARED((V, L), jnp.float32),
            pltpu.SemaphoreType.DMA,
        ],
        compiler_params=pltpu.CompilerParams(use_tc_tiling_on_sc=False),
    )
    return f(emb, idx3, zeros, ones)


def _final_body(x_ref, q_ref, cnt_ref, qst_ref, loss_ref, perp_ref):
    xv = x_ref[...]
    qv = q_ref[...]
    diff = qv - xv
    qst_ref[...] = xv + diff
    loss = 0.25 * (jnp.sum(diff * diff) * (1.0 / (B * D)))
    loss_ref[...] = jnp.reshape(loss, (1, 1))
    cnt = cnt_ref[0:1, :] + cnt_ref[1:2, :]
    p = cnt * (1.0 / B)
    s = jnp.sum(p * jnp.log(p + 1e-10))
    perp_ref[...] = jnp.reshape(jnp.exp(-s), (1, 1))


def _final_call(x, quant, cnt2):
    return pl.pallas_call(
        _final_body,
        out_shape=[
            jax.ShapeDtypeStruct((B, D), jnp.float32),
            jax.ShapeDtypeStruct((1, 1), jnp.float32),
            jax.ShapeDtypeStruct((1, 1), jnp.float32),
        ],
    )(x, quant, cnt2)


def kernel(inputs, emb_weight):
    emb_t = emb_weight.T
    idx = _argmin_call(inputs, emb_t)
    quant, counts_parts = _sc_call(emb_weight, idx)
    cnt2 = counts_parts.reshape(NC, V, L)[:, :, 0]
    qst, loss, perp = _final_call(inputs, quant, cnt2)
    return (qst, jnp.reshape(loss, ()), idx.reshape(B, 1),
            jnp.reshape(perp, ()))
